# baseline (speedup 1.0000x reference)
import functools

import jax
import jax.numpy as jnp
from jax import lax
from jax.experimental import pallas as pl
from jax.experimental.pallas import tpu as pltpu
from jax.experimental.pallas import tpu_sc as plsc

N_OBJ = 10000
N_EDGE = 160000
N_CLS = 151
HID = 256
N_LAYERS = 4
ROWT = 1000
GRID = N_OBJ // ROWT

_NC = 2
_NS = 16
_L = 16
_NW = _NC * _NS
_EPW = 5008
_EPAD = _EPW * _NW

_SC_MESH = dict(core_axis_name="c", subcore_axis_name="s",
                num_cores=_NC, num_subcores=_NS)


def _edge_w_body(preds_hbm, tab_hbm, src_hbm, dst_hbm, w_hbm,
                 preds_v, tab_v, src_v, dst_v, w_v):
    c = lax.axis_index("c")
    s = lax.axis_index("s")
    wid = s * _NC + c
    base = wid * _EPW
    pltpu.sync_copy(preds_hbm, preds_v)
    pltpu.sync_copy(tab_hbm, tab_v)
    pltpu.sync_copy(src_hbm.at[pl.ds(base, _EPW)], src_v)
    pltpu.sync_copy(dst_hbm.at[pl.ds(base, _EPW)], dst_v)

    def body(i, carry):
        sl = pl.ds(i * _L, _L)
        ps = plsc.load_gather(preds_v, [src_v[sl]])
        pd = plsc.load_gather(preds_v, [dst_v[sl]])
        w_v[sl] = plsc.load_gather(tab_v, [ps, pd])
        return carry

    lax.fori_loop(0, _EPW // _L, body, 0)
    pltpu.sync_copy(w_v, w_hbm.at[pl.ds(base, _EPW)])


def _edge_weights(preds, tabw, src_pad, dst_pad):
    k = pl.kernel(
        _edge_w_body,
        out_type=jax.ShapeDtypeStruct((_EPAD,), jnp.float32),
        mesh=plsc.VectorSubcoreMesh(**_SC_MESH),
        compiler_params=pltpu.CompilerParams(needs_layout_passes=False),
        scratch_types=[
            pltpu.VMEM((N_OBJ,), jnp.int32),
            pltpu.VMEM((N_CLS, N_CLS), jnp.float32),
            pltpu.VMEM((_EPW,), jnp.int32),
            pltpu.VMEM((_EPW,), jnp.int32),
            pltpu.VMEM((_EPW,), jnp.float32),
        ],
    )
    return k(preds, tabw, src_pad, dst_pad)


def _enc_body(box_f_ref, w1_ref, b1_ref, g_ref, bb_ref, w2_ref, b2_ref,
              x_ref, lg_ref, box_t_ref, oew_ref, wcx_ref, wce_ref, wcp_ref,
              bctx_ref, wout_ref, bout_ref,
              repa_ref, repb_ref, preds_ref):
    h = box_f_ref[...] @ w1_ref[...] + b1_ref[...]
    mu = jnp.mean(h, axis=0, keepdims=True)
    var = jnp.mean((h - mu) ** 2, axis=0, keepdims=True)
    scale = g_ref[...] * lax.rsqrt(var + 1e-5)
    shift = bb_ref[...] - mu * scale
    ht = box_t_ref[...] @ w1_ref[...] + b1_ref[...]
    pos = jnp.maximum((ht * scale + shift) @ w2_ref[...] + b2_ref[...], 0.0)
    lg = lg_ref[...]
    p = jax.nn.softmax(lg, axis=-1)
    e = p @ oew_ref[...]
    rep = (x_ref[...] @ wcx_ref[...] + e @ wce_ref[...] + pos @ wcp_ref[...]
           + bctx_ref[...])
    d0 = rep @ wout_ref[...] + bout_ref[...]
    col = lax.broadcasted_iota(jnp.int32, d0.shape, 1)
    d0m = jnp.where(col > 0, d0, -jnp.inf)
    pr = jnp.argmax(d0m, axis=-1).astype(jnp.int32)
    repa_ref[...] = rep[:, :128]
    repb_ref[...] = rep[:, 128:]
    preds_ref[...] = pr[:, None]


def _encoder(box_info, pos_w1, pos_b1, pos_bn_g, pos_bn_b, pos_w2, pos_b2,
             x, obj_logits, obj_embed_w, W_ctx, b_ctx, W_out, b_out):
    full = lambda shape: pl.BlockSpec(shape, lambda i: (0, 0))
    tile = lambda cols: pl.BlockSpec((ROWT, cols), lambda i: (i, 0))
    out = pl.pallas_call(
        _enc_body,
        grid=(GRID,),
        in_specs=[
            full((N_OBJ, 9)), full((9, 32)), full((1, 32)), full((1, 32)),
            full((1, 32)), full((32, 128)), full((1, 128)),
            tile(128), tile(N_CLS), tile(9), full((N_CLS, 200)),
            full((128, HID)), full((200, HID)), full((128, HID)),
            full((1, HID)), full((HID, N_CLS)), full((1, N_CLS)),
        ],
        out_specs=[tile(128), tile(128),
                   pl.BlockSpec((ROWT, 1), lambda i: (i, 0))],
        out_shape=[
            jax.ShapeDtypeStruct((N_OBJ, 128), jnp.float32),
            jax.ShapeDtypeStruct((N_OBJ, 128), jnp.float32),
            jax.ShapeDtypeStruct((N_OBJ, 1), jnp.int32),
        ],
    )(box_info, pos_w1, pos_b1.reshape(1, -1), pos_bn_g.reshape(1, -1),
      pos_bn_b.reshape(1, -1), pos_w2, pos_b2.reshape(1, -1),
      x, obj_logits, box_info, obj_embed_w,
      W_ctx[:128], W_ctx[128:328], W_ctx[328:], b_ctx.reshape(1, -1),
      W_out, b_out.reshape(1, -1))
    repa, repb, preds2 = out
    return repa, repb, preds2[:, 0]


_CH = 128
_NCT = 86
_ETILE = _NCT * _CH
_NPAD = 10240
_RPT = _NPAD // _NS


def _agg_body(repa_hbm, repb_hbm, pk_hbm, agga_hbm, aggb_hbm,
              pk0, pk1, pk2, rw0, rw1, agg_sh,
              ps0, ps1, ps2, gs0, gs1, ss0, ss1):
    c = lax.axis_index("c")
    s = lax.axis_index("s")
    pks = [pk0, pk1, pk2]
    rows = [rw0, rw1]
    psems = [ps0, ps1, ps2]
    gsems = [gs0, gs1]
    ssems = [ss0, ss1]

    def _pack_start(i, b):
        pltpu.async_copy(pk_hbm.at[s, i], pks[b], psems[b])

    def _pack_wait(b):
        pltpu.make_async_copy(pk_hbm.at[s, 0], pks[b], psems[b]).wait()

    def _gather_start(rb, b):
        idx = pks[b].at[0]

        @pl.when(c == 0)
        def _():
            pltpu.async_copy(repa_hbm.at[idx], rows[rb], gsems[rb])

        @pl.when(c == 1)
        def _():
            pltpu.async_copy(repb_hbm.at[idx], rows[rb], gsems[rb])

    def _gather_wait(rb, b):
        idx = pks[b].at[0]

        @pl.when(c == 0)
        def _():
            pltpu.make_async_copy(repa_hbm.at[idx], rows[rb],
                                  gsems[rb]).wait()

        @pl.when(c == 1)
        def _():
            pltpu.make_async_copy(repb_hbm.at[idx], rows[rb],
                                  gsems[rb]).wait()

    def _scale_scatter(rb, b):
        two = jnp.full((_L,), 2, jnp.int32)
        rw = rows[rb]
        pk = pks[b]

        def scale(i4, carry):
            for r4 in range(4):
                e = i4 * 4 + r4
                wv = plsc.bitcast(
                    plsc.load_gather(
                        pk, [two, jnp.full((_L,), e, jnp.int32)]),
                    jnp.float32)
                for r in range(8):
                    sl = pl.ds(r * _L, _L)
                    rw[e, sl] = rw[e, sl] * wv
            return carry
        lax.fori_loop(0, _CH // 4, scale, 0)
        pltpu.async_copy(rw, agg_sh.at[pk.at[1]], ssems[rb], add=True)

    def _scatter_wait(rb, b):
        pltpu.make_async_copy(rows[rb], agg_sh.at[pks[b].at[1]],
                              ssems[rb]).wait()

    def zrow(j, carry):
        for r in range(8):
            rw0[j, pl.ds(r * _L, _L)] = jnp.zeros((_L,), jnp.float32)
        return carry
    lax.fori_loop(0, _CH, zrow, 0)

    def zcopy(k2, carry):
        pltpu.sync_copy(rw0, agg_sh.at[pl.ds(s * _RPT + k2 * _CH, _CH)])
        return carry
    lax.fori_loop(0, _RPT // _CH, zcopy, 0)
    plsc.subcore_barrier()

    def _emit(ires, iact):
        _gather_wait(ires % 2, ires % 3)
        if ires >= 1:
            _scatter_wait((ires - 1) % 2, (ires - 1) % 3)
        if ires <= _NCT - 3:
            _pack_start(iact + 2, (ires + 2) % 3)
        if ires <= _NCT - 2:
            _pack_wait((ires + 1) % 3)
            _gather_start((ires + 1) % 2, (ires + 1) % 3)
        _scale_scatter(ires % 2, ires % 3)

    _pack_start(0, 0)
    _pack_start(1, 1)
    _pack_wait(0)
    _gather_start(0, 0)
    _emit(0, 0)
    _emit(1, 1)

    def outer(k6, carry):
        g = 2 + k6 * 6
        for q in range(6):
            _emit(2 + q, g + q)
        return carry
    lax.fori_loop(0, (_NCT - 8) // 6, outer, 0, unroll=False)

    for i in range(_NCT - 6, _NCT):
        _emit(i, i)
    _scatter_wait((_NCT - 1) % 2, (_NCT - 1) % 3)
    plsc.subcore_barrier()

    def drain(k2, carry):
        off = s * _RPT + k2 * _CH
        pltpu.sync_copy(agg_sh.at[pl.ds(off, _CH)], rw0)

        @pl.when(c == 0)
        def _():
            pltpu.sync_copy(rw0, agga_hbm.at[pl.ds(off, _CH)])

        @pl.when(c == 1)
        def _():
            pltpu.sync_copy(rw0, aggb_hbm.at[pl.ds(off, _CH)])
        return carry
    lax.fori_loop(0, _RPT // _CH, drain, 0)


def _aggregate(repa, repb, packed):
    k = pl.kernel(
        _agg_body,
        out_type=[jax.ShapeDtypeStruct((_NPAD, 128), jnp.float32)] * 2,
        mesh=plsc.VectorSubcoreMesh(**_SC_MESH),
        compiler_params=pltpu.CompilerParams(needs_layout_passes=False),
        scratch_types=(
            [pltpu.VMEM((3, _CH), jnp.int32)] * 3
            + [pltpu.VMEM((_CH, 128), jnp.float32)] * 2
            + [pltpu.VMEM_SHARED((_NPAD, 128), jnp.float32)]
            + [pltpu.SemaphoreType.DMA] * 7),
    )
    return k(repa, repb, packed)


def _tab_body(fb_ref, out_ref):
    fb = fb_ref[...]
    m = jnp.max(fb, axis=-1, keepdims=True)
    s = jnp.sum(jnp.exp(fb - m), axis=-1, keepdims=True)
    out_ref[...] = 1.0 / s


def _pair_table(freq_bias):
    out = pl.pallas_call(
        _tab_body,
        out_shape=jax.ShapeDtypeStruct((N_CLS * N_CLS, 1), jnp.float32),
    )(freq_bias)
    return out.reshape(N_CLS, N_CLS)


def _upd_body(agga_ref, aggb_ref, repa_ref, repb_ref, wt_ref, wb_ref,
              newa_ref, newb_ref):
    o = jnp.maximum(agga_ref[...] @ wt_ref[...] + aggb_ref[...] @ wb_ref[...],
                    0.0)
    newa_ref[...] = o[:, :128] + repa_ref[...]
    newb_ref[...] = o[:, 128:] + repb_ref[...]


def _update(agga, aggb, repa, repb, wm):
    tile = pl.BlockSpec((ROWT, 128), lambda i: (i, 0))
    full = lambda shape: pl.BlockSpec(shape, lambda i: (0, 0))
    return pl.pallas_call(
        _upd_body,
        grid=(GRID,),
        in_specs=[tile, tile, tile, tile, full((128, HID)), full((128, HID))],
        out_specs=[tile, tile],
        out_shape=[jax.ShapeDtypeStruct((N_OBJ, 128), jnp.float32)] * 2,
    )(agga, aggb, repa, repb, wm[:128], wm[128:])


def _final_body(agga_ref, aggb_ref, repa_ref, repb_ref, wt_ref, wb_ref,
                wouta_ref, woutb_ref, bout_ref, dists_ref):
    o = jnp.maximum(agga_ref[...] @ wt_ref[...] + aggb_ref[...] @ wb_ref[...],
                    0.0)
    na = o[:, :128] + repa_ref[...]
    nb = o[:, 128:] + repb_ref[...]
    dists_ref[...] = na @ wouta_ref[...] + nb @ woutb_ref[...] + bout_ref[...]


def _final(agga, aggb, repa, repb, wm, W_out, b_out):
    tile = pl.BlockSpec((ROWT, 128), lambda i: (i, 0))
    full = lambda shape: pl.BlockSpec(shape, lambda i: (0, 0))
    return pl.pallas_call(
        _final_body,
        grid=(GRID,),
        in_specs=[tile, tile, tile, tile, full((128, HID)), full((128, HID)),
                  full((128, N_CLS)), full((128, N_CLS)), full((1, N_CLS))],
        out_specs=pl.BlockSpec((ROWT, N_CLS), lambda i: (i, 0)),
        out_shape=jax.ShapeDtypeStruct((N_OBJ, N_CLS), jnp.float32),
    )(agga, aggb, repa, repb, wm[:128], wm[128:],
      W_out[:128], W_out[128:], b_out.reshape(1, -1))


def kernel(x, obj_logits, box_info, rel_pair_idxs, freq_bias, obj_embed_w,
           pos_w1, pos_b1, pos_bn_g, pos_bn_b, pos_w2, pos_b2,
           W_ctx, b_ctx, W_out, b_out, W_lin, b_lin, W_msg):
    repa, repb, preds = _encoder(box_info, pos_w1, pos_b1, pos_bn_g,
                                 pos_bn_b, pos_w2, pos_b2, x, obj_logits,
                                 obj_embed_w, W_ctx, b_ctx, W_out, b_out)
    tabw = _pair_table(freq_bias)
    src = rel_pair_idxs[0]
    dst = rel_pair_idxs[1]
    pad = jnp.zeros((_EPAD - N_EDGE,), jnp.int32)
    src_pad = jnp.concatenate([src.astype(jnp.int32), pad])
    dst_pad = jnp.concatenate([dst.astype(jnp.int32), pad])
    w = _edge_weights(preds, tabw, src_pad, dst_pad)[:N_EDGE]
    src2d = src.astype(jnp.int32).reshape(_NS, N_EDGE // _NS)
    dst2d = dst.astype(jnp.int32).reshape(_NS, N_EDGE // _NS)
    w2d = lax.bitcast_convert_type(w, jnp.int32).reshape(_NS, N_EDGE // _NS)
    padn = _ETILE - N_EDGE // _NS
    zpad = jnp.zeros((_NS, padn), jnp.int32)
    src2d = jnp.concatenate([src2d, zpad], axis=1)
    dst2d = jnp.concatenate([dst2d, zpad + 10200], axis=1)
    w2d = jnp.concatenate([w2d, zpad], axis=1)
    packed = jnp.stack([src2d.reshape(_NS, _NCT, _CH),
                        dst2d.reshape(_NS, _NCT, _CH),
                        w2d.reshape(_NS, _NCT, _CH)], axis=2)
    for i in range(N_LAYERS):
        agga, aggb = _aggregate(repa, repb, packed)
        agga = agga[:N_OBJ]
        aggb = aggb[:N_OBJ]
        if i < N_LAYERS - 1:
            repa, repb = _update(agga, aggb, repa, repb, W_msg[i])
        else:
            return _final(agga, aggb, repa, repb, W_msg[i], W_out, b_out)


# speedup vs baseline: 1.9253x; 1.9253x over previous
import functools

import jax
import jax.numpy as jnp
from jax import lax
from jax.experimental import pallas as pl
from jax.experimental.pallas import tpu as pltpu
from jax.experimental.pallas import tpu_sc as plsc

N_OBJ = 10000
N_EDGE = 160000
N_CLS = 151
HID = 256
N_LAYERS = 4
ROWT = 1000
GRID = N_OBJ // ROWT

_NC = 2
_NS = 16
_L = 16
_NW = _NC * _NS
_EPW = 5008
_EPAD = _EPW * _NW

_SC_MESH = dict(core_axis_name="c", subcore_axis_name="s",
                num_cores=_NC, num_subcores=_NS)


def _edge_w_body(preds_hbm, tab_hbm, src_hbm, dst_hbm, w_hbm,
                 preds_v, tab_v, src_v, dst_v, w_v):
    c = lax.axis_index("c")
    s = lax.axis_index("s")
    wid = s * _NC + c
    base = wid * _EPW
    pltpu.sync_copy(preds_hbm, preds_v)
    pltpu.sync_copy(tab_hbm, tab_v)
    pltpu.sync_copy(src_hbm.at[pl.ds(base, _EPW)], src_v)
    pltpu.sync_copy(dst_hbm.at[pl.ds(base, _EPW)], dst_v)

    def body(i, carry):
        sl = pl.ds(i * _L, _L)
        ps = plsc.load_gather(preds_v, [src_v[sl]])
        pd = plsc.load_gather(preds_v, [dst_v[sl]])
        w_v[sl] = plsc.load_gather(tab_v, [ps, pd])
        return carry

    lax.fori_loop(0, _EPW // _L, body, 0)
    pltpu.sync_copy(w_v, w_hbm.at[pl.ds(base, _EPW)])


def _edge_weights(preds, tabw, src_pad, dst_pad):
    k = pl.kernel(
        _edge_w_body,
        out_type=jax.ShapeDtypeStruct((_EPAD,), jnp.float32),
        mesh=plsc.VectorSubcoreMesh(**_SC_MESH),
        compiler_params=pltpu.CompilerParams(needs_layout_passes=False),
        scratch_types=[
            pltpu.VMEM((N_OBJ,), jnp.int32),
            pltpu.VMEM((N_CLS, N_CLS), jnp.float32),
            pltpu.VMEM((_EPW,), jnp.int32),
            pltpu.VMEM((_EPW,), jnp.int32),
            pltpu.VMEM((_EPW,), jnp.float32),
        ],
    )
    return k(preds, tabw, src_pad, dst_pad)


def _enc_body(box_f_ref, w1_ref, b1_ref, g_ref, bb_ref, w2_ref, b2_ref,
              x_ref, lg_ref, box_t_ref, oew_ref, wcx_ref, wce_ref, wcp_ref,
              bctx_ref, wout_ref, bout_ref,
              repa_ref, repb_ref, preds_ref):
    h = box_f_ref[...] @ w1_ref[...] + b1_ref[...]
    mu = jnp.mean(h, axis=0, keepdims=True)
    var = jnp.mean((h - mu) ** 2, axis=0, keepdims=True)
    scale = g_ref[...] * lax.rsqrt(var + 1e-5)
    shift = bb_ref[...] - mu * scale
    ht = box_t_ref[...] @ w1_ref[...] + b1_ref[...]
    pos = jnp.maximum((ht * scale + shift) @ w2_ref[...] + b2_ref[...], 0.0)
    lg = lg_ref[...]
    p = jax.nn.softmax(lg, axis=-1)
    e = p @ oew_ref[...]
    rep = (x_ref[...] @ wcx_ref[...] + e @ wce_ref[...] + pos @ wcp_ref[...]
           + bctx_ref[...])
    d0 = rep @ wout_ref[...] + bout_ref[...]
    col = lax.broadcasted_iota(jnp.int32, d0.shape, 1)
    d0m = jnp.where(col > 0, d0, -jnp.inf)
    pr = jnp.argmax(d0m, axis=-1).astype(jnp.int32)
    repa_ref[...] = rep[:, :128]
    repb_ref[...] = rep[:, 128:]
    preds_ref[...] = pr[:, None]


def _encoder(box_info, pos_w1, pos_b1, pos_bn_g, pos_bn_b, pos_w2, pos_b2,
             x, obj_logits, obj_embed_w, W_ctx, b_ctx, W_out, b_out):
    full = lambda shape: pl.BlockSpec(shape, lambda i: (0, 0))
    tile = lambda cols: pl.BlockSpec((ROWT, cols), lambda i: (i, 0))
    out = pl.pallas_call(
        _enc_body,
        grid=(GRID,),
        in_specs=[
            full((N_OBJ, 9)), full((9, 32)), full((1, 32)), full((1, 32)),
            full((1, 32)), full((32, 128)), full((1, 128)),
            tile(128), tile(N_CLS), tile(9), full((N_CLS, 200)),
            full((128, HID)), full((200, HID)), full((128, HID)),
            full((1, HID)), full((HID, N_CLS)), full((1, N_CLS)),
        ],
        out_specs=[tile(128), tile(128),
                   pl.BlockSpec((ROWT, 1), lambda i: (i, 0))],
        out_shape=[
            jax.ShapeDtypeStruct((N_OBJ, 128), jnp.float32),
            jax.ShapeDtypeStruct((N_OBJ, 128), jnp.float32),
            jax.ShapeDtypeStruct((N_OBJ, 1), jnp.int32),
        ],
    )(box_info, pos_w1, pos_b1.reshape(1, -1), pos_bn_g.reshape(1, -1),
      pos_bn_b.reshape(1, -1), pos_w2, pos_b2.reshape(1, -1),
      x, obj_logits, box_info, obj_embed_w,
      W_ctx[:128], W_ctx[128:328], W_ctx[328:], b_ctx.reshape(1, -1),
      W_out, b_out.reshape(1, -1))
    repa, repb, preds2 = out
    return repa, repb, preds2[:, 0]


_CH = 64
_NCT = 162
_ETILE = _NCT * _CH
_NPAD = 10240
_RPT = _NPAD // _NS


def _agg_body(repa_hbm, repb_hbm, pk_hbm, agga_hbm, aggb_hbm,
              pk0, pk1, pk2, pk3, rw0, rw1, rw2, agg_sh,
              ps0, ps1, ps2, ps3, gs0, gs1, gs2, ss0, ss1, ss2):
    c = lax.axis_index("c")
    s = lax.axis_index("s")
    pks = [pk0, pk1, pk2, pk3]
    rows = [rw0, rw1, rw2]
    psems = [ps0, ps1, ps2, ps3]
    gsems = [gs0, gs1, gs2]
    ssems = [ss0, ss1, ss2]

    def _pack_start(i, b):
        pltpu.async_copy(pk_hbm.at[s, i], pks[b], psems[b])

    def _pack_wait(b):
        pltpu.make_async_copy(pk_hbm.at[s, 0], pks[b], psems[b]).wait()

    def _gather_start(rb, b):
        idx = pks[b].at[0]

        @pl.when(c == 0)
        def _():
            pltpu.async_copy(repa_hbm.at[idx], rows[rb], gsems[rb])

        @pl.when(c == 1)
        def _():
            pltpu.async_copy(repb_hbm.at[idx], rows[rb], gsems[rb])

    def _gather_wait(rb, b):
        idx = pks[b].at[0]

        @pl.when(c == 0)
        def _():
            pltpu.make_async_copy(repa_hbm.at[idx], rows[rb],
                                  gsems[rb]).wait()

        @pl.when(c == 1)
        def _():
            pltpu.make_async_copy(repb_hbm.at[idx], rows[rb],
                                  gsems[rb]).wait()

    def _scale_scatter(rb, b):
        two = jnp.full((_L,), 2, jnp.int32)
        rw = rows[rb]
        pk = pks[b]

        def scale(i4, carry):
            for r4 in range(4):
                e = i4 * 4 + r4
                wv = plsc.bitcast(
                    plsc.load_gather(
                        pk, [two, jnp.full((_L,), e, jnp.int32)]),
                    jnp.float32)
                for r in range(8):
                    sl = pl.ds(r * _L, _L)
                    rw[e, sl] = rw[e, sl] * wv
            return carry
        lax.fori_loop(0, _CH // 4, scale, 0)
        pltpu.async_copy(rw, agg_sh.at[pk.at[1]], ssems[rb], add=True)

    def _scatter_wait(rb, b):
        pltpu.make_async_copy(rows[rb], agg_sh.at[pks[b].at[1]],
                              ssems[rb]).wait()

    def zrow(j, carry):
        for r in range(8):
            rw0[j, pl.ds(r * _L, _L)] = jnp.zeros((_L,), jnp.float32)
        return carry
    lax.fori_loop(0, _CH, zrow, 0)

    def zcopy(k2, carry):
        pltpu.sync_copy(rw0, agg_sh.at[pl.ds(s * _RPT + k2 * _CH, _CH)])
        return carry
    lax.fori_loop(0, _RPT // _CH, zcopy, 0)
    plsc.subcore_barrier()

    def _emit(ires, iact):
        if ires >= 2:
            _scatter_wait((ires - 2) % 3, (ires - 2) % 4)
        if ires <= _NCT - 3:
            _pack_start(iact + 2, (ires + 2) % 4)
        if ires <= _NCT - 2:
            _pack_wait((ires + 1) % 4)
            _gather_start((ires + 1) % 3, (ires + 1) % 4)
        _gather_wait(ires % 3, ires % 4)
        _scale_scatter(ires % 3, ires % 4)

    _pack_start(0, 0)
    _pack_start(1, 1)
    _pack_wait(0)
    _gather_start(0, 0)
    _emit(0, 0)
    _emit(1, 1)
    _emit(2, 2)

    def outer(k12, carry):
        g = 3 + k12 * 12
        for q in range(12):
            _emit(3 + q, g + q)
        return carry
    lax.fori_loop(0, (_NCT - 6) // 12, outer, 0, unroll=False)

    for i in range(_NCT - 3, _NCT):
        _emit(i, i)
    _scatter_wait((_NCT - 2) % 3, (_NCT - 2) % 4)
    _scatter_wait((_NCT - 1) % 3, (_NCT - 1) % 4)
    plsc.subcore_barrier()

    def drain(k2, carry):
        off = s * _RPT + k2 * _CH
        pltpu.sync_copy(agg_sh.at[pl.ds(off, _CH)], rw0)

        @pl.when(c == 0)
        def _():
            pltpu.sync_copy(rw0, agga_hbm.at[pl.ds(off, _CH)])

        @pl.when(c == 1)
        def _():
            pltpu.sync_copy(rw0, aggb_hbm.at[pl.ds(off, _CH)])
        return carry
    lax.fori_loop(0, _RPT // _CH, drain, 0)


def _aggregate(repa, repb, packed):
    k = pl.kernel(
        _agg_body,
        out_type=[jax.ShapeDtypeStruct((_NPAD, 128), jnp.float32)] * 2,
        mesh=plsc.VectorSubcoreMesh(**_SC_MESH),
        compiler_params=pltpu.CompilerParams(needs_layout_passes=False),
        scratch_types=(
            [pltpu.VMEM((3, _CH), jnp.int32)] * 4
            + [pltpu.VMEM((_CH, 128), jnp.float32)] * 3
            + [pltpu.VMEM_SHARED((_NPAD, 128), jnp.float32)]
            + [pltpu.SemaphoreType.DMA] * 10),
    )
    return k(repa, repb, packed)


def _tab_body(fb_ref, out_ref):
    fb = fb_ref[...]
    m = jnp.max(fb, axis=-1, keepdims=True)
    s = jnp.sum(jnp.exp(fb - m), axis=-1, keepdims=True)
    out_ref[...] = 1.0 / s


def _pair_table(freq_bias):
    out = pl.pallas_call(
        _tab_body,
        out_shape=jax.ShapeDtypeStruct((N_CLS * N_CLS, 1), jnp.float32),
    )(freq_bias)
    return out.reshape(N_CLS, N_CLS)


def _upd_body(agga_ref, aggb_ref, repa_ref, repb_ref, wt_ref, wb_ref,
              newa_ref, newb_ref):
    o = jnp.maximum(agga_ref[...] @ wt_ref[...] + aggb_ref[...] @ wb_ref[...],
                    0.0)
    newa_ref[...] = o[:, :128] + repa_ref[...]
    newb_ref[...] = o[:, 128:] + repb_ref[...]


def _update(agga, aggb, repa, repb, wm):
    tile = pl.BlockSpec((ROWT, 128), lambda i: (i, 0))
    full = lambda shape: pl.BlockSpec(shape, lambda i: (0, 0))
    return pl.pallas_call(
        _upd_body,
        grid=(GRID,),
        in_specs=[tile, tile, tile, tile, full((128, HID)), full((128, HID))],
        out_specs=[tile, tile],
        out_shape=[jax.ShapeDtypeStruct((N_OBJ, 128), jnp.float32)] * 2,
    )(agga, aggb, repa, repb, wm[:128], wm[128:])


def _final_body(agga_ref, aggb_ref, repa_ref, repb_ref, wt_ref, wb_ref,
                wouta_ref, woutb_ref, bout_ref, dists_ref):
    o = jnp.maximum(agga_ref[...] @ wt_ref[...] + aggb_ref[...] @ wb_ref[...],
                    0.0)
    na = o[:, :128] + repa_ref[...]
    nb = o[:, 128:] + repb_ref[...]
    dists_ref[...] = na @ wouta_ref[...] + nb @ woutb_ref[...] + bout_ref[...]


def _final(agga, aggb, repa, repb, wm, W_out, b_out):
    tile = pl.BlockSpec((ROWT, 128), lambda i: (i, 0))
    full = lambda shape: pl.BlockSpec(shape, lambda i: (0, 0))
    return pl.pallas_call(
        _final_body,
        grid=(GRID,),
        in_specs=[tile, tile, tile, tile, full((128, HID)), full((128, HID)),
                  full((128, N_CLS)), full((128, N_CLS)), full((1, N_CLS))],
        out_specs=pl.BlockSpec((ROWT, N_CLS), lambda i: (i, 0)),
        out_shape=jax.ShapeDtypeStruct((N_OBJ, N_CLS), jnp.float32),
    )(agga, aggb, repa, repb, wm[:128], wm[128:],
      W_out[:128], W_out[128:], b_out.reshape(1, -1))


def kernel(x, obj_logits, box_info, rel_pair_idxs, freq_bias, obj_embed_w,
           pos_w1, pos_b1, pos_bn_g, pos_bn_b, pos_w2, pos_b2,
           W_ctx, b_ctx, W_out, b_out, W_lin, b_lin, W_msg):
    repa, repb, preds = _encoder(box_info, pos_w1, pos_b1, pos_bn_g,
                                 pos_bn_b, pos_w2, pos_b2, x, obj_logits,
                                 obj_embed_w, W_ctx, b_ctx, W_out, b_out)
    tabw = _pair_table(freq_bias)
    src = rel_pair_idxs[0]
    dst = rel_pair_idxs[1]
    pad = jnp.zeros((_EPAD - N_EDGE,), jnp.int32)
    src_pad = jnp.concatenate([src.astype(jnp.int32), pad])
    dst_pad = jnp.concatenate([dst.astype(jnp.int32), pad])
    w = _edge_weights(preds, tabw, src_pad, dst_pad)[:N_EDGE]
    src2d = src.astype(jnp.int32).reshape(_NS, N_EDGE // _NS)
    dst2d = dst.astype(jnp.int32).reshape(_NS, N_EDGE // _NS)
    w2d = lax.bitcast_convert_type(w, jnp.int32).reshape(_NS, N_EDGE // _NS)
    padn = _ETILE - N_EDGE // _NS
    zpad = jnp.zeros((_NS, padn), jnp.int32)
    src2d = jnp.concatenate([src2d, zpad], axis=1)
    dst2d = jnp.concatenate([dst2d, zpad + 10200], axis=1)
    w2d = jnp.concatenate([w2d, zpad], axis=1)
    packed = jnp.stack([src2d.reshape(_NS, _NCT, _CH),
                        dst2d.reshape(_NS, _NCT, _CH),
                        w2d.reshape(_NS, _NCT, _CH)], axis=2)
    for i in range(N_LAYERS):
        agga, aggb = _aggregate(repa, repb, packed)
        agga = agga[:N_OBJ]
        aggb = aggb[:N_OBJ]
        if i < N_LAYERS - 1:
            repa, repb = _update(agga, aggb, repa, repb, W_msg[i])
        else:
            return _final(agga, aggb, repa, repb, W_msg[i], W_out, b_out)
